# baseline (speedup 1.0000x reference)
import functools

import jax
import jax.numpy as jnp
from jax import lax
from jax.experimental import pallas as pl
from jax.experimental.pallas import tpu as pltpu
from jax.experimental.pallas import tpu_sc as plsc

D_MODEL = 64
SCALE = 8.0
BT = 128
NB = 5
STP = BT + 1

B_DIM = 4096
L_DIM = 200
LT = L_DIM // 8
BTG = B_DIM // BT
N_SG = LT * BTG


@functools.lru_cache(maxsize=None)
def _make_kernel():
    info = plsc.get_sparse_core_info()
    nc, ns, lanes = info.num_cores, info.num_subcores, info.num_lanes
    nw = nc * ns
    assert N_SG % nw == 0
    sg_per_w = N_SG // nw
    n_k = sg_per_w * 8
    assert n_k % NB == 0
    vpr = D_MODEL // lanes

    mesh = plsc.VectorSubcoreMesh(core_axis_name="c", subcore_axis_name="s")

    @functools.partial(
        pl.kernel,
        mesh=mesh,
        out_type=jax.ShapeDtypeStruct((L_DIM, 8, BTG, 8, BT), jnp.float32),
        scratch_types=[
            pltpu.VMEM((2, 8, BT), jnp.int32),
            pltpu.VMEM((NB, BT, D_MODEL), jnp.float32),
            pltpu.VMEM((NB, 8, 8, STP), jnp.float32),
        ]
        + [pltpu.SemaphoreType.DMA] * (2 + 2 * NB),
        compiler_params=pltpu.CompilerParams(
            use_tc_tiling_on_sc=False, needs_layout_passes=False
        ),
    )
    def emb_kernel(table_hbm, xs_hbm, out_hbm, idx_v, rows, st, *sems):
        isems = sems[:2]
        gsems = sems[2 : 2 + NB]
        ssems = sems[2 + NB :]
        wid = lax.axis_index("s") * nc + lax.axis_index("c")
        sg0 = wid * sg_per_w

        dts = [
            (jnp.arange(lanes, dtype=jnp.int32) + c * lanes) >> 3
            for c in range(vpr)
        ]
        drs = [
            (jnp.arange(lanes, dtype=jnp.int32) + c * lanes) & 7
            for c in range(vpr)
        ]

        def idx_desc(sg, ib):
            return pltpu.make_async_copy(
                xs_hbm.at[sg // BTG, sg % BTG], idx_v.at[ib], isems[ib]
            )

        def gather_desc(ib, lr, rb):
            return pltpu.make_async_copy(
                table_hbm.at[idx_v.at[ib, lr]], rows.at[rb], gsems[rb]
            )

        def store_descs(k, sb):
            sg = sg0 + k // 8
            l = (sg // BTG) * 8 + (k % 8)
            bt = sg % BTG
            return [
                pltpu.make_async_copy(
                    st.at[sb, :, :, pl.ds(0, BT)],
                    out_hbm.at[l, :, bt],
                    ssems[sb],
                )
            ]

        def transpose_scale(rb, sb):
            st1 = st.at[sb]

            @plsc.parallel_loop(0, BT, unroll=16)
            def body(br):
                vb = jnp.full((lanes,), br, jnp.int32)
                for c in range(vpr):
                    vals = rows[rb, br, pl.ds(c * lanes, lanes)] * SCALE
                    plsc.store_scatter(st1, [dts[c], drs[c], vb], vals)

        idx_desc(sg0, 0).start()
        idx_desc(sg0 + 1, 1).start()
        idx_desc(sg0, 0).wait()
        for k in range(NB - 1):
            gather_desc(0, k, k).start()

        def round_body(r, carry):
            for u in range(NB):
                k = r * NB + u
                m = k // 8
                lr = k % 8
                ib = m % 2
                gather_desc(ib, lr, u).wait()
                @pl.when(jnp.logical_and(lr == 7, m + 2 < sg_per_w))
                def _():
                    for ib2 in range(2):
                        @pl.when(ib == ib2)
                        def _():
                            idx_desc(sg0 + m + 2, ib2).start()
                @pl.when(r > 0)
                def _():
                    for d_ in store_descs(k - NB, u):
                        d_.wait()
                transpose_scale(u, u)
                for d_ in store_descs(k, u):
                    d_.start()
                kf = k + NB - 1
                @pl.when(kf < n_k)
                def _():
                    mf = kf // 8
                    lrf = kf % 8
                    ibf = mf % 2
                    @pl.when(lrf == 0)
                    def _():
                        for ib2 in range(2):
                            @pl.when(ibf == ib2)
                            def _():
                                idx_desc(sg0 + mf, ib2).wait()
                    gather_desc(ibf, lrf, (u + NB - 1) % NB).start()
            return carry

        lax.fori_loop(0, n_k // NB, round_body, 0)
        for u in range(NB):
            for d_ in store_descs(n_k - NB + u, u):
                d_.wait()

    return emb_kernel


def kernel(x, table):
    xs = (
        x.astype(jnp.int32)
        .T.reshape(LT, 8, BTG, BT)
        .transpose(0, 2, 1, 3)
    )
    out5 = _make_kernel()(table, xs)
    out = out5.transpose(2, 4, 0, 1, 3).reshape(B_DIM, L_DIM, D_MODEL)
    return out


# speedup vs baseline: 1.0197x; 1.0197x over previous
import functools

import jax
import jax.numpy as jnp
from jax import lax
from jax.experimental import pallas as pl
from jax.experimental.pallas import tpu as pltpu
from jax.experimental.pallas import tpu_sc as plsc

D_MODEL = 64
SCALE = 8.0
BT = 128
NB = 5
STP = BT + 1

B_DIM = 4096
L_DIM = 200
LT = L_DIM // 8
BTG = B_DIM // BT
N_SG = LT * BTG


@functools.lru_cache(maxsize=None)
def _make_kernel():
    info = plsc.get_sparse_core_info()
    nc, ns, lanes = info.num_cores, info.num_subcores, info.num_lanes
    nw = nc * ns
    assert N_SG % nw == 0
    sg_per_w = N_SG // nw
    n_k = sg_per_w * 8
    assert n_k % NB == 0
    vpr = D_MODEL // lanes

    mesh = plsc.VectorSubcoreMesh(core_axis_name="c", subcore_axis_name="s")

    @functools.partial(
        pl.kernel,
        mesh=mesh,
        out_type=jax.ShapeDtypeStruct((L_DIM, 8, BTG, 8, BT), jnp.float32),
        scratch_types=[
            pltpu.VMEM((2, 8, BT), jnp.int32),
            pltpu.VMEM((NB, BT, D_MODEL), jnp.float32),
            pltpu.VMEM((NB, 8, 8, STP), jnp.float32),
        ]
        + [pltpu.SemaphoreType.DMA] * (2 + 2 * NB),
        compiler_params=pltpu.CompilerParams(
            use_tc_tiling_on_sc=False, needs_layout_passes=False
        ),
    )
    def emb_kernel(table_hbm, xs_hbm, out_hbm, idx_v, rows, st, *sems):
        isems = sems[:2]
        gsems = sems[2 : 2 + NB]
        ssems = sems[2 + NB :]
        wid = lax.axis_index("s") * nc + lax.axis_index("c")
        sg0 = wid * sg_per_w

        dts = [
            (jnp.arange(lanes, dtype=jnp.int32) + c * lanes) >> 3
            for c in range(vpr)
        ]
        drs = [
            (jnp.arange(lanes, dtype=jnp.int32) + c * lanes) & 7
            for c in range(vpr)
        ]

        def idx_desc(sg, ib):
            return pltpu.make_async_copy(
                xs_hbm.at[sg // BTG, sg % BTG], idx_v.at[ib], isems[ib]
            )

        def gather_desc(ib, lr, rb):
            return pltpu.make_async_copy(
                table_hbm.at[idx_v.at[ib, lr]], rows.at[rb], gsems[rb]
            )

        def store_descs(k, sb):
            sg = sg0 + k // 8
            l = (sg // BTG) * 8 + (k % 8)
            bt = sg % BTG
            return [
                pltpu.make_async_copy(
                    st.at[sb, :, :, pl.ds(0, BT)],
                    out_hbm.at[l, :, bt],
                    ssems[sb],
                )
            ]

        def transpose_scale(rb, sb):
            st1 = st.at[sb]

            @plsc.parallel_loop(0, BT, unroll=8)
            def body(br):
                vb = jnp.full((lanes,), br, jnp.int32)
                for c in range(vpr):
                    vals = rows[rb, br, pl.ds(c * lanes, lanes)] * SCALE
                    plsc.store_scatter(st1, [dts[c], drs[c], vb], vals)

        idx_desc(sg0, 0).start()
        idx_desc(sg0 + 1, 1).start()
        idx_desc(sg0, 0).wait()
        for k in range(NB - 1):
            gather_desc(0, k, k).start()

        def round_body(r, carry):
            for u in range(NB):
                k = r * NB + u
                m = k // 8
                lr = k % 8
                ib = m % 2
                gather_desc(ib, lr, u).wait()
                @pl.when(jnp.logical_and(lr == 7, m + 2 < sg_per_w))
                def _():
                    for ib2 in range(2):
                        @pl.when(ib == ib2)
                        def _():
                            idx_desc(sg0 + m + 2, ib2).start()
                @pl.when(r > 0)
                def _():
                    for d_ in store_descs(k - NB, u):
                        d_.wait()
                transpose_scale(u, u)
                for d_ in store_descs(k, u):
                    d_.start()
                kf = k + NB - 1
                @pl.when(kf < n_k)
                def _():
                    mf = kf // 8
                    lrf = kf % 8
                    ibf = mf % 2
                    @pl.when(lrf == 0)
                    def _():
                        for ib2 in range(2):
                            @pl.when(ibf == ib2)
                            def _():
                                idx_desc(sg0 + mf, ib2).wait()
                    gather_desc(ibf, lrf, (u + NB - 1) % NB).start()
            return carry

        lax.fori_loop(0, n_k // NB, round_body, 0)
        for u in range(NB):
            for d_ in store_descs(n_k - NB + u, u):
                d_.wait()

    return emb_kernel


def kernel(x, table):
    xs = (
        x.astype(jnp.int32)
        .T.reshape(LT, 8, BTG, BT)
        .transpose(0, 2, 1, 3)
    )
    out5 = _make_kernel()(table, xs)
    out = out5.transpose(2, 4, 0, 1, 3).reshape(B_DIM, L_DIM, D_MODEL)
    return out
